# Initial kernel scaffold; baseline (speedup 1.0000x reference)
import jax
import jax.numpy as jnp
from jax.experimental import pallas as pl


def kernel(x, edge_index, batch, emb_table, W1, b1, W2, b2, W3, b3, Wfc, bfc):
    raise NotImplementedError("write your pallas kernel here")


# speedup vs baseline: 7.6852x; 7.6852x over previous
import functools
import jax
import jax.numpy as jnp
from jax import lax
from jax.experimental import pallas as pl
from jax.experimental.pallas import tpu as pltpu
from jax.experimental.pallas import tpu_sc as plsc

_N = 10000
_E = 320000
_D_IN = 128
_EMB = 64
_HID = 128
_HH = 64
_NPAD = 10240

_NC = 2
_NS = 16
_NW = _NC * _NS
_EPW = _E // _NW
_ECH = 1000
_CCH = 400
_NROW = _NPAD // _NW
_RPT = _NPAD // _NS

_mesh = plsc.VectorSubcoreMesh(core_axis_name="c", subcore_axis_name="s")
_F32 = jnp.float32


def _wid():
    return lax.axis_index("s") * _NC + lax.axis_index("c")


@functools.partial(
    pl.kernel,
    out_type=jax.ShapeDtypeStruct((_NPAD, _EMB), _F32),
    mesh=_mesh,
    scratch_types=[
        pltpu.VMEM((_NROW,), jnp.int32),
        pltpu.VMEM((_NROW, _EMB), _F32),
        pltpu.SemaphoreType.DMA,
    ],
    compiler_params=pltpu.CompilerParams(use_tc_tiling_on_sc=False),
)
def _emb_gather(table_hbm, idx_hbm, out_hbm, idx_v, rows_v, sem):
    base = _wid() * _NROW
    pltpu.sync_copy(idx_hbm.at[pl.ds(base, _NROW)], idx_v)
    pltpu.async_copy(table_hbm.at[idx_v], rows_v, sem).wait()
    pltpu.sync_copy(rows_v, out_hbm.at[pl.ds(base, _NROW)])


_DCH = 2000


@functools.partial(
    pl.kernel,
    out_type=jax.ShapeDtypeStruct((_NC, _NPAD), _F32),
    mesh=_mesh,
    scratch_types=[
        pltpu.VMEM((_DCH,), jnp.int32),
        pltpu.VMEM((_DCH,), _F32),
        pltpu.VMEM_SHARED((_NPAD,), _F32),
    ],
)
def _degrees(dst_hbm, out_hbm, idx_v, ones_v, acc_sh):
    c = lax.axis_index("c")
    s = lax.axis_index("s")
    w = s * _NC + c

    def fill0(i, _):
        ones_v[pl.ds(i * 16, 16)] = jnp.zeros((16,), _F32)
        return 0

    lax.fori_loop(0, _DCH // 16, fill0, 0)
    pltpu.sync_copy(ones_v.at[pl.ds(0, _RPT)], acc_sh.at[pl.ds(s * _RPT, _RPT)])

    def fill1(i, _):
        ones_v[pl.ds(i * 16, 16)] = jnp.ones((16,), _F32)
        return 0

    lax.fori_loop(0, _DCH // 16, fill1, 0)
    plsc.subcore_barrier()

    def step(i, _):
        base = w * _EPW + i * _DCH
        pltpu.sync_copy(dst_hbm.at[pl.ds(base, _DCH)], idx_v)
        pltpu.sync_copy(ones_v, acc_sh.at[idx_v], add=True)
        return 0

    lax.fori_loop(0, _EPW // _DCH, step, 0)
    plsc.subcore_barrier()
    pltpu.sync_copy(acc_sh.at[pl.ds(s * _RPT, _RPT)],
                    out_hbm.at[c, pl.ds(s * _RPT, _RPT)])


@functools.partial(
    pl.kernel,
    out_type=jax.ShapeDtypeStruct((_NC, _NPAD, _HH), _F32),
    mesh=_mesh,
    scratch_types=[
        pltpu.VMEM((_ECH,), jnp.int32),
        pltpu.VMEM((_ECH,), jnp.int32),
        pltpu.VMEM((_ECH, _HH), _F32),
        pltpu.VMEM_SHARED((_NPAD, _HH), _F32),
        pltpu.SemaphoreType.DMA,
    ],
    compiler_params=pltpu.CompilerParams(use_tc_tiling_on_sc=False),
)
def _propagate_half(hs_hbm, src_hbm, dst_hbm, out_hbm, sidx_v, didx_v, rows_v,
                    acc_sh, sem):
    c = lax.axis_index("c")
    s = lax.axis_index("s")
    w = s * _NC + c

    def zfill(i, _):
        for j in range(_HH // 16):
            rows_v[i, pl.ds(j * 16, 16)] = jnp.zeros((16,), _F32)
        return 0

    lax.fori_loop(0, _RPT, zfill, 0)
    pltpu.sync_copy(rows_v.at[pl.ds(0, _RPT)],
                    acc_sh.at[pl.ds(s * _RPT, _RPT)])
    plsc.subcore_barrier()

    def step(i, _):
        base = w * _EPW + i * _ECH
        pltpu.sync_copy(src_hbm.at[pl.ds(base, _ECH)], sidx_v)
        pltpu.sync_copy(dst_hbm.at[pl.ds(base, _ECH)], didx_v)
        pltpu.async_copy(hs_hbm.at[sidx_v], rows_v, sem).wait()
        pltpu.sync_copy(rows_v, acc_sh.at[didx_v], add=True)
        return 0

    lax.fori_loop(0, _EPW // _ECH, step, 0)
    plsc.subcore_barrier()
    pltpu.sync_copy(acc_sh.at[pl.ds(s * _RPT, _RPT)],
                    out_hbm.at[c, pl.ds(s * _RPT, _RPT)])


@functools.partial(
    pl.kernel,
    out_type=jax.ShapeDtypeStruct((_E, _HID), _F32),
    mesh=_mesh,
    scratch_types=[
        pltpu.VMEM((_CCH,), jnp.int32),
        pltpu.VMEM((_CCH,), jnp.int32),
        pltpu.VMEM((_CCH, _HID), _F32),
        pltpu.VMEM((_CCH, _HID), _F32),
        pltpu.SemaphoreType.DMA,
    ],
)
def _edge_combine(a_hbm, b_hbm, src_hbm, dst_hbm, out_hbm, sidx_v, didx_v,
                  abuf_v, bbuf_v, sem):
    w = _wid()

    def step(i, _):
        base = w * _EPW + i * _CCH
        pltpu.sync_copy(src_hbm.at[pl.ds(base, _CCH)], sidx_v)
        pltpu.sync_copy(dst_hbm.at[pl.ds(base, _CCH)], didx_v)
        pltpu.async_copy(a_hbm.at[sidx_v], abuf_v, sem).wait()
        pltpu.async_copy(b_hbm.at[didx_v], bbuf_v, sem).wait()

        def add_row(r, _):
            for j in range(_HID // 16):
                sl = pl.ds(j * 16, 16)
                abuf_v[r, sl] = abuf_v[r, sl] + bbuf_v[r, sl]
            return 0

        lax.fori_loop(0, _CCH, add_row, 0)
        pltpu.sync_copy(abuf_v, out_hbm.at[pl.ds(base, _CCH)])
        return 0

    lax.fori_loop(0, _EPW // _CCH, step, 0)


def _gelu(v):
    return 0.5 * v * (1.0 + lax.erf(v * 0.7071067811865476))


def _dinv_of(degt_ref):
    return lax.rsqrt(degt_ref[:, 0:1] + degt_ref[:, 1:2] + 1.0)


def _d1_body(h_ref, w_ref, degt_ref, outl_ref, outr_ref):
    dinv = _dinv_of(degt_ref)
    hs = dinv * jnp.dot(h_ref[...], w_ref[...], preferred_element_type=_F32)
    outl_ref[...] = hs[:, :_HH]
    outr_ref[...] = hs[:, _HH:]


def _mid_body(rl0_ref, rl1_ref, rr0_ref, rr1_ref, hsl_ref, hsr_ref, degt_ref,
              b_ref, w_ref, outl_ref, outr_ref):
    dinv = _dinv_of(degt_ref)
    agg = jnp.concatenate(
        [rl0_ref[...] + rl1_ref[...] + hsl_ref[...],
         rr0_ref[...] + rr1_ref[...] + hsr_ref[...]], axis=1)
    g = _gelu(dinv * agg + b_ref[...])
    hs = dinv * jnp.dot(g, w_ref[...], preferred_element_type=_F32)
    outl_ref[...] = hs[:, :_HH]
    outr_ref[...] = hs[:, _HH:]


def _last_body(rl0_ref, rl1_ref, rr0_ref, rr1_ref, hsl_ref, hsr_ref, degt_ref,
               b_ref, wa_ref, wb_ref, bfc_ref, outa_ref, outb_ref):
    dinv = _dinv_of(degt_ref)
    agg = jnp.concatenate(
        [rl0_ref[...] + rl1_ref[...] + hsl_ref[...],
         rr0_ref[...] + rr1_ref[...] + hsr_ref[...]], axis=1)
    g = _gelu(dinv * agg + b_ref[...])
    outa_ref[...] = jnp.dot(g, wa_ref[...],
                            preferred_element_type=_F32) + bfc_ref[...]
    outb_ref[...] = jnp.dot(g, wb_ref[...], preferred_element_type=_F32)


def _gelu_body(s_ref, out_ref):
    out_ref[...] = _gelu(s_ref[...])


_NSPEC = pl.BlockSpec((_NPAD, _HID), lambda: (0, 0))
_HSPEC = pl.BlockSpec((_NPAD, _HH), lambda: (0, 0))
_WSPEC = pl.BlockSpec((_HID, _HID), lambda: (0, 0))


def _half_shapes():
    return (jax.ShapeDtypeStruct((_NPAD, _HH), _F32),
            jax.ShapeDtypeStruct((_NPAD, _HH), _F32))


def kernel(x, edge_index, batch, emb_table, W1, b1, W2, b2, W3, b3, Wfc, bfc):
    src = edge_index[0]
    dst = edge_index[1]

    node_idx = jnp.pad(x[:, -1].astype(jnp.int32), (0, _NPAD - _N))
    emb = _emb_gather(emb_table, node_idx)
    degp = _degrees(dst)
    degt = jnp.transpose(degp)

    feats = jnp.pad(x[:, :_D_IN], ((0, _NPAD - _N), (0, 0)))
    h0 = jnp.concatenate([feats, emb], axis=1)

    dspec = pl.BlockSpec((_NPAD, 2), lambda: (0, 0))
    bspec = pl.BlockSpec((1, _HID), lambda: (0, 0))

    hsl, hsr = pl.pallas_call(
        _d1_body,
        out_shape=_half_shapes(),
        in_specs=[
            pl.BlockSpec((_NPAD, _D_IN + _EMB), lambda: (0, 0)),
            pl.BlockSpec((_D_IN + _EMB, _HID), lambda: (0, 0)),
            dspec,
        ],
        out_specs=(_HSPEC, _HSPEC),
    )(h0, W1, degt)

    def mid_layer(hsl, hsr, W, b):
        rl = _propagate_half(hsl, src, dst)
        rr = _propagate_half(hsr, src, dst)
        return pl.pallas_call(
            _mid_body,
            out_shape=_half_shapes(),
            in_specs=[_HSPEC] * 6 + [dspec, bspec, _WSPEC],
            out_specs=(_HSPEC, _HSPEC),
        )(rl[0], rl[1], rr[0], rr[1], hsl, hsr, degt, b[None, :], W)

    hsl, hsr = mid_layer(hsl, hsr, W2, b1)
    hsl, hsr = mid_layer(hsl, hsr, W3, b2)

    rl = _propagate_half(hsl, src, dst)
    rr = _propagate_half(hsr, src, dst)
    A, B = pl.pallas_call(
        _last_body,
        out_shape=(jax.ShapeDtypeStruct((_NPAD, _HID), _F32),
                   jax.ShapeDtypeStruct((_NPAD, _HID), _F32)),
        in_specs=[_HSPEC] * 6 + [dspec, bspec, _WSPEC, _WSPEC, bspec],
        out_specs=(_NSPEC, _NSPEC),
    )(rl[0], rl[1], rr[0], rr[1], hsl, hsr, degt, b3[None, :],
      Wfc[:_HID], Wfc[_HID:], bfc[None, :])

    S = _edge_combine(A, B, src, dst)

    _EB = 4000
    out = pl.pallas_call(
        _gelu_body,
        grid=(_E // _EB,),
        out_shape=jax.ShapeDtypeStruct((_E, _HID), _F32),
        in_specs=[pl.BlockSpec((_EB, _HID), lambda i: (i, 0))],
        out_specs=pl.BlockSpec((_EB, _HID), lambda i: (i, 0)),
    )(S)
    return out


# speedup vs baseline: 8.2137x; 1.0688x over previous
import functools
import jax
import jax.numpy as jnp
from jax import lax
from jax.experimental import pallas as pl
from jax.experimental.pallas import tpu as pltpu
from jax.experimental.pallas import tpu_sc as plsc

_N = 10000
_E = 320000
_D_IN = 128
_EMB = 64
_HID = 128
_HH = 64
_NPAD = 10240

_NC = 2
_NS = 16
_NW = _NC * _NS
_EPW = _E // _NW
_ECH = 200
_CCH = 200
_NROW = _NPAD // _NW
_RPT = _NPAD // _NS

_mesh = plsc.VectorSubcoreMesh(core_axis_name="c", subcore_axis_name="s")
_F32 = jnp.float32


def _wid():
    return lax.axis_index("s") * _NC + lax.axis_index("c")


@functools.partial(
    pl.kernel,
    out_type=jax.ShapeDtypeStruct((_NPAD, _EMB), _F32),
    mesh=_mesh,
    scratch_types=[
        pltpu.VMEM((_NROW,), jnp.int32),
        pltpu.VMEM((_NROW, _EMB), _F32),
        pltpu.SemaphoreType.DMA,
    ],
    compiler_params=pltpu.CompilerParams(use_tc_tiling_on_sc=False),
)
def _emb_gather(table_hbm, idx_hbm, out_hbm, idx_v, rows_v, sem):
    base = _wid() * _NROW
    pltpu.sync_copy(idx_hbm.at[pl.ds(base, _NROW)], idx_v)
    pltpu.async_copy(table_hbm.at[idx_v], rows_v, sem).wait()
    pltpu.sync_copy(rows_v, out_hbm.at[pl.ds(base, _NROW)])


_DCH = 2000


@functools.partial(
    pl.kernel,
    out_type=jax.ShapeDtypeStruct((_NC, _NPAD), _F32),
    mesh=_mesh,
    scratch_types=[
        pltpu.VMEM((_DCH,), jnp.int32),
        pltpu.VMEM((_DCH,), _F32),
        pltpu.VMEM_SHARED((_NPAD,), _F32),
    ],
)
def _degrees(dst_hbm, out_hbm, idx_v, ones_v, acc_sh):
    c = lax.axis_index("c")
    s = lax.axis_index("s")
    w = s * _NC + c

    def fill0(i, _):
        ones_v[pl.ds(i * 16, 16)] = jnp.zeros((16,), _F32)
        return 0

    lax.fori_loop(0, _DCH // 16, fill0, 0)
    pltpu.sync_copy(ones_v.at[pl.ds(0, _RPT)], acc_sh.at[pl.ds(s * _RPT, _RPT)])

    def fill1(i, _):
        ones_v[pl.ds(i * 16, 16)] = jnp.ones((16,), _F32)
        return 0

    lax.fori_loop(0, _DCH // 16, fill1, 0)
    plsc.subcore_barrier()

    def step(i, _):
        base = w * _EPW + i * _DCH
        pltpu.sync_copy(dst_hbm.at[pl.ds(base, _DCH)], idx_v)
        pltpu.sync_copy(ones_v, acc_sh.at[idx_v], add=True)
        return 0

    lax.fori_loop(0, _EPW // _DCH, step, 0)
    plsc.subcore_barrier()
    pltpu.sync_copy(acc_sh.at[pl.ds(s * _RPT, _RPT)],
                    out_hbm.at[c, pl.ds(s * _RPT, _RPT)])


_PNCH = _EPW // _ECH


@functools.partial(
    pl.kernel,
    out_type=jax.ShapeDtypeStruct((_NC, _NPAD, _HH), _F32),
    mesh=_mesh,
    scratch_types=[
        pltpu.VMEM((_ECH,), jnp.int32),
        pltpu.VMEM((_ECH,), jnp.int32),
        pltpu.VMEM((_ECH,), jnp.int32),
        pltpu.VMEM((_ECH, _HH), _F32),
        pltpu.VMEM((_ECH, _HH), _F32),
        pltpu.VMEM_SHARED((_NPAD, _HH), _F32),
        pltpu.SemaphoreType.DMA,
        pltpu.SemaphoreType.DMA,
    ],
    compiler_params=pltpu.CompilerParams(use_tc_tiling_on_sc=False),
)
def _propagate_half(hs_hbm, src_hbm, dst_hbm, out_hbm, sidx0_v, sidx1_v,
                    didx_v, rows0_v, rows1_v, acc_sh, sem0, sem1):
    c = lax.axis_index("c")
    s = lax.axis_index("s")
    w = s * _NC + c
    ebase = w * _EPW
    rows = (rows0_v, rows1_v)
    sidx = (sidx0_v, sidx1_v)
    sems = (sem0, sem1)

    def zfill(i, _):
        for j in range(_HH // 16):
            rows0_v[i, pl.ds(j * 16, 16)] = jnp.zeros((16,), _F32)
        return 0

    lax.fori_loop(0, 160, zfill, 0)
    for r in range(_RPT // 160):
        pltpu.sync_copy(rows0_v.at[pl.ds(0, 160)],
                        acc_sh.at[pl.ds(s * _RPT + r * 160, 160)])
    plsc.subcore_barrier()

    pltpu.sync_copy(src_hbm.at[pl.ds(ebase, _ECH)], sidx0_v)
    pltpu.async_copy(hs_hbm.at[sidx0_v], rows0_v, sem0)

    def pair(t, _):
        for p in range(2):
            i = 2 * t + p
            q = 1 - p

            @pl.when((i + 1) < _PNCH)
            def _issue():
                nb = ebase + (i + 1) * _ECH
                pltpu.sync_copy(src_hbm.at[pl.ds(nb, _ECH)], sidx[q])
                pltpu.async_copy(hs_hbm.at[sidx[q]], rows[q], sems[q])

            pltpu.make_async_copy(hs_hbm.at[sidx[p]], rows[p],
                                  sems[p]).wait()
            pltpu.sync_copy(dst_hbm.at[pl.ds(ebase + i * _ECH, _ECH)], didx_v)
            pltpu.sync_copy(rows[p], acc_sh.at[didx_v], add=True)
        return 0

    lax.fori_loop(0, _PNCH // 2, pair, 0)
    plsc.subcore_barrier()
    pltpu.sync_copy(acc_sh.at[pl.ds(s * _RPT, _RPT)],
                    out_hbm.at[c, pl.ds(s * _RPT, _RPT)])


_CNCH = _EPW // _CCH


@functools.partial(
    pl.kernel,
    out_type=jax.ShapeDtypeStruct((_E, _HID), _F32),
    mesh=_mesh,
    scratch_types=[
        pltpu.VMEM((_CCH,), jnp.int32),
        pltpu.VMEM((_CCH,), jnp.int32),
        pltpu.VMEM((_CCH,), jnp.int32),
        pltpu.VMEM((_CCH,), jnp.int32),
        pltpu.VMEM((_CCH, _HID), _F32),
        pltpu.VMEM((_CCH, _HID), _F32),
        pltpu.VMEM((_CCH, _HID), _F32),
        pltpu.VMEM((_CCH, _HID), _F32),
        pltpu.SemaphoreType.DMA,
        pltpu.SemaphoreType.DMA,
        pltpu.SemaphoreType.DMA,
        pltpu.SemaphoreType.DMA,
    ],
)
def _edge_combine(a_hbm, b_hbm, src_hbm, dst_hbm, out_hbm, sidx0_v, sidx1_v,
                  didx0_v, didx1_v, abuf0_v, abuf1_v, bbuf0_v, bbuf1_v,
                  sema0, sema1, semb0, semb1):
    w = _wid()
    ebase = w * _EPW
    abufs = (abuf0_v, abuf1_v)
    bbufs = (bbuf0_v, bbuf1_v)
    sidx = (sidx0_v, sidx1_v)
    didx = (didx0_v, didx1_v)
    semas = (sema0, sema1)
    sembs = (semb0, semb1)

    def issue(i, q):
        nb = ebase + i * _CCH
        pltpu.sync_copy(src_hbm.at[pl.ds(nb, _CCH)], sidx[q])
        pltpu.sync_copy(dst_hbm.at[pl.ds(nb, _CCH)], didx[q])
        pltpu.async_copy(a_hbm.at[sidx[q]], abufs[q], semas[q])
        pltpu.async_copy(b_hbm.at[didx[q]], bbufs[q], sembs[q])

    issue(0, 0)

    def pair(t, _):
        for p in range(2):
            i = 2 * t + p
            q = 1 - p

            @pl.when((i + 1) < _CNCH)
            def _issue():
                issue(i + 1, q)

            pltpu.make_async_copy(a_hbm.at[sidx[p]], abufs[p],
                                  semas[p]).wait()
            pltpu.make_async_copy(b_hbm.at[didx[p]], bbufs[p],
                                  sembs[p]).wait()

            def add_row(r, _):
                for j in range(_HID // 16):
                    sl = pl.ds(j * 16, 16)
                    abufs[p][r, sl] = abufs[p][r, sl] + bbufs[p][r, sl]
                return 0

            lax.fori_loop(0, _CCH, add_row, 0)
            pltpu.sync_copy(abufs[p], out_hbm.at[pl.ds(ebase + i * _CCH,
                                                       _CCH)])
        return 0

    lax.fori_loop(0, _CNCH // 2, pair, 0)


def _gelu(v):
    return 0.5 * v * (1.0 + lax.erf(v * 0.7071067811865476))


def _dinv_of(degt_ref):
    return lax.rsqrt(degt_ref[:, 0:1] + degt_ref[:, 1:2] + 1.0)


def _d1_body(h_ref, w_ref, degt_ref, outl_ref, outr_ref):
    dinv = _dinv_of(degt_ref)
    hs = dinv * jnp.dot(h_ref[...], w_ref[...], preferred_element_type=_F32)
    outl_ref[...] = hs[:, :_HH]
    outr_ref[...] = hs[:, _HH:]


def _mid_body(rl0_ref, rl1_ref, rr0_ref, rr1_ref, hsl_ref, hsr_ref, degt_ref,
              bl_ref, br_ref, wt_ref, wb_ref, outl_ref, outr_ref):
    dinv = _dinv_of(degt_ref)
    gl = _gelu(dinv * (rl0_ref[...] + rl1_ref[...] + hsl_ref[...])
               + bl_ref[...])
    gr = _gelu(dinv * (rr0_ref[...] + rr1_ref[...] + hsr_ref[...])
               + br_ref[...])
    hs = dinv * (jnp.dot(gl, wt_ref[...], preferred_element_type=_F32)
                 + jnp.dot(gr, wb_ref[...], preferred_element_type=_F32))
    outl_ref[...] = hs[:, :_HH]
    outr_ref[...] = hs[:, _HH:]


def _last_body(rl0_ref, rl1_ref, rr0_ref, rr1_ref, hsl_ref, hsr_ref, degt_ref,
               bl_ref, br_ref, wat_ref, wab_ref, wbt_ref, wbb_ref, bfc_ref,
               outa_ref, outb_ref):
    dinv = _dinv_of(degt_ref)
    gl = _gelu(dinv * (rl0_ref[...] + rl1_ref[...] + hsl_ref[...])
               + bl_ref[...])
    gr = _gelu(dinv * (rr0_ref[...] + rr1_ref[...] + hsr_ref[...])
               + br_ref[...])
    outa_ref[...] = (jnp.dot(gl, wat_ref[...], preferred_element_type=_F32)
                     + jnp.dot(gr, wab_ref[...], preferred_element_type=_F32)
                     + bfc_ref[...])
    outb_ref[...] = (jnp.dot(gl, wbt_ref[...], preferred_element_type=_F32)
                     + jnp.dot(gr, wbb_ref[...], preferred_element_type=_F32))


def _gelu_body(s_ref, out_ref):
    out_ref[...] = _gelu(s_ref[...])


_BN = 2048
_NG = _NPAD // _BN
_NSPEC = pl.BlockSpec((_BN, _HID), lambda i: (i, 0))
_HSPEC = pl.BlockSpec((_BN, _HH), lambda i: (i, 0))


def _half_shapes():
    return (jax.ShapeDtypeStruct((_NPAD, _HH), _F32),
            jax.ShapeDtypeStruct((_NPAD, _HH), _F32))


def kernel(x, edge_index, batch, emb_table, W1, b1, W2, b2, W3, b3, Wfc, bfc):
    src = edge_index[0]
    dst = edge_index[1]

    node_idx = jnp.pad(x[:, -1].astype(jnp.int32), (0, _NPAD - _N))
    emb = _emb_gather(emb_table, node_idx)
    degp = _degrees(dst)
    degt = jnp.transpose(degp)

    feats = jnp.pad(x[:, :_D_IN], ((0, _NPAD - _N), (0, 0)))
    h0 = jnp.concatenate([feats, emb], axis=1)

    dspec = pl.BlockSpec((_BN, 2), lambda i: (i, 0))
    bspec = pl.BlockSpec((1, _HID), lambda i: (0, 0))

    hsl, hsr = pl.pallas_call(
        _d1_body,
        grid=(_NG,),
        out_shape=_half_shapes(),
        in_specs=[
            pl.BlockSpec((_BN, _D_IN + _EMB), lambda i: (i, 0)),
            pl.BlockSpec((_D_IN + _EMB, _HID), lambda i: (0, 0)),
            dspec,
        ],
        out_specs=(_HSPEC, _HSPEC),
    )(h0, W1, degt)

    hbspec = pl.BlockSpec((1, _HH), lambda i: (0, 0))
    hwspec = pl.BlockSpec((_HH, _HID), lambda i: (0, 0))

    def mid_layer(hsl, hsr, W, b):
        rl = _propagate_half(hsl, src, dst)
        rr = _propagate_half(hsr, src, dst)
        return pl.pallas_call(
            _mid_body,
            grid=(_NG,),
            out_shape=_half_shapes(),
            in_specs=[_HSPEC] * 6 + [dspec, hbspec, hbspec, hwspec, hwspec],
            out_specs=(_HSPEC, _HSPEC),
        )(rl[0], rl[1], rr[0], rr[1], hsl, hsr, degt,
          b[None, :_HH], b[None, _HH:], W[:_HH], W[_HH:])

    hsl, hsr = mid_layer(hsl, hsr, W2, b1)
    hsl, hsr = mid_layer(hsl, hsr, W3, b2)

    rl = _propagate_half(hsl, src, dst)
    rr = _propagate_half(hsr, src, dst)
    A, B = pl.pallas_call(
        _last_body,
        grid=(_NG,),
        out_shape=(jax.ShapeDtypeStruct((_NPAD, _HID), _F32),
                   jax.ShapeDtypeStruct((_NPAD, _HID), _F32)),
        in_specs=[_HSPEC] * 6 + [dspec, hbspec, hbspec, hwspec, hwspec,
                                 hwspec, hwspec, bspec],
        out_specs=(_NSPEC, _NSPEC),
    )(rl[0], rl[1], rr[0], rr[1], hsl, hsr, degt,
      b3[None, :_HH], b3[None, _HH:],
      Wfc[:_HH], Wfc[_HH:_HID], Wfc[_HID:_HID + _HH], Wfc[_HID + _HH:],
      bfc[None, :])

    S = _edge_combine(A, B, src, dst)

    _EB = 4000
    out = pl.pallas_call(
        _gelu_body,
        grid=(_E // _EB,),
        out_shape=jax.ShapeDtypeStruct((_E, _HID), _F32),
        in_specs=[pl.BlockSpec((_EB, _HID), lambda i: (i, 0))],
        out_specs=pl.BlockSpec((_EB, _HID), lambda i: (i, 0)),
    )(S)
    return out
